# Initial kernel scaffold; baseline (speedup 1.0000x reference)
import jax
import jax.numpy as jnp
from jax.experimental import pallas as pl


def kernel(src_x, src_edge_index, src_edge_mask, src_batch, src_mask, tgt_x, tgt_edge_index, tgt_edge_mask, tgt_batch, We0, ase0, ade0, be0, We1, ase1, ade1, be1, Wd0, asd0, add0, bd0, Wd1, asd1, add1, bd1):
    raise NotImplementedError("write your pallas kernel here")


# speedup vs baseline: 5.1395x; 5.1395x over previous
import functools
import jax
import jax.numpy as jnp
from jax.experimental import pallas as pl

_N_BLK = 1000
_E_BLK = 8000


def _mm_body(x_ref, w_ref, o_ref):
    o_ref[...] = jnp.dot(x_ref[...], w_ref[...],
                         preferred_element_type=jnp.float32)


def _node_matmul(x, w):
    n, din = x.shape
    f = w.shape[1]
    grid = (pl.cdiv(n, _N_BLK),)
    return pl.pallas_call(
        _mm_body,
        grid=grid,
        in_specs=[
            pl.BlockSpec((_N_BLK, din), lambda i: (i, 0)),
            pl.BlockSpec((din, f), lambda i: (0, 0)),
        ],
        out_specs=pl.BlockSpec((_N_BLK, f), lambda i: (i, 0)),
        out_shape=jax.ShapeDtypeStruct((n, f), jnp.float32),
    )(x, w)


def _edge_body(asrc_ref, adst_ref, hsrc_ref, r_ref, ex_ref, w_ref):
    a = asrc_ref[...] + adst_ref[...]
    a = jnp.where(a >= 0.0, a, 0.2 * a)
    ex = jnp.exp(a)
    ex_ref[...] = ex
    exrep = jnp.dot(ex, r_ref[...], preferred_element_type=jnp.float32)
    w_ref[...] = exrep * hsrc_ref[...]


def _edge_coef(asrc_g, adst_g, hsrc_g, rmat):
    e, h = asrc_g.shape
    hc = hsrc_g.shape[1]
    grid = (pl.cdiv(e, _E_BLK),)
    return pl.pallas_call(
        _edge_body,
        grid=grid,
        in_specs=[
            pl.BlockSpec((_E_BLK, h), lambda i: (i, 0)),
            pl.BlockSpec((_E_BLK, h), lambda i: (i, 0)),
            pl.BlockSpec((_E_BLK, hc), lambda i: (i, 0)),
            pl.BlockSpec((h, hc), lambda i: (0, 0)),
        ],
        out_specs=[
            pl.BlockSpec((_E_BLK, h), lambda i: (i, 0)),
            pl.BlockSpec((_E_BLK, hc), lambda i: (i, 0)),
        ],
        out_shape=[
            jax.ShapeDtypeStruct((e, h), jnp.float32),
            jax.ShapeDtypeStruct((e, hc), jnp.float32),
        ],
    )(asrc_g, adst_g, hsrc_g, rmat)


def _epilogue_body(relu, accw_ref, den_ref, asrc_ref, adst_ref, h_ref,
                   r_ref, b_ref, o_ref):
    a = asrc_ref[...] + adst_ref[...]
    a = jnp.where(a >= 0.0, a, 0.2 * a)
    exl = jnp.exp(a)
    dtot = jnp.maximum(den_ref[...] + exl, 1e-16)
    inv = 1.0 / dtot
    r = r_ref[...]
    exlrep = jnp.dot(exl, r, preferred_element_type=jnp.float32)
    invrep = jnp.dot(inv, r, preferred_element_type=jnp.float32)
    out = (accw_ref[...] + exlrep * h_ref[...]) * invrep + b_ref[...]
    if relu:
        out = jnp.maximum(out, 0.0)
    o_ref[...] = out


def _epilogue(accw, den, asrc, adst, h, rmat, b, relu):
    n, hc = accw.shape
    nh = den.shape[1]
    grid = (pl.cdiv(n, _N_BLK),)
    return pl.pallas_call(
        functools.partial(_epilogue_body, relu),
        grid=grid,
        in_specs=[
            pl.BlockSpec((_N_BLK, hc), lambda i: (i, 0)),
            pl.BlockSpec((_N_BLK, nh), lambda i: (i, 0)),
            pl.BlockSpec((_N_BLK, nh), lambda i: (i, 0)),
            pl.BlockSpec((_N_BLK, nh), lambda i: (i, 0)),
            pl.BlockSpec((_N_BLK, hc), lambda i: (i, 0)),
            pl.BlockSpec((nh, hc), lambda i: (0, 0)),
            pl.BlockSpec((1, hc), lambda i: (0, 0)),
        ],
        out_specs=pl.BlockSpec((_N_BLK, hc), lambda i: (i, 0)),
        out_shape=jax.ShapeDtypeStruct((n, hc), jnp.float32),
    )(accw, den, asrc, adst, h, rmat, b)


def _gat_layer(x, src, dst, W, a_src, a_dst, b, H, C, relu):
    n = x.shape[0]
    hc = H * C
    eye = jnp.eye(H, dtype=jnp.float32)
    amat_s = (a_src[0][:, :, None] * eye[:, None, :]).reshape(hc, H)
    amat_d = (a_dst[0][:, :, None] * eye[:, None, :]).reshape(hc, H)
    rmat = jnp.repeat(eye, C, axis=1)
    wfull = jnp.concatenate([W, W @ amat_s, W @ amat_d], axis=1)
    hfull = _node_matmul(x, wfull)
    h = hfull[:, :hc]
    asrc = hfull[:, hc:hc + H]
    adst = hfull[:, hc + H:]
    asrc_g = jnp.take(asrc, src, axis=0)
    adst_g = jnp.take(adst, dst, axis=0)
    hsrc_g = jnp.take(h, src, axis=0)
    ex, wsum = _edge_coef(asrc_g, adst_g, hsrc_g, rmat)
    den = jax.ops.segment_sum(ex, dst, num_segments=n)
    accw = jax.ops.segment_sum(wsum, dst, num_segments=n)
    return _epilogue(accw, den, asrc, adst, h, rmat, b.reshape(1, hc), relu)


def kernel(src_x, src_edge_index, src_edge_mask, src_batch, src_mask,
           tgt_x, tgt_edge_index, tgt_edge_mask, tgt_batch,
           We0, ase0, ade0, be0, We1, ase1, ade1, be1,
           Wd0, asd0, add0, bd0, Wd1, asd1, add1, bd1):
    B = 64
    s_src, s_dst = src_edge_index[0], src_edge_index[1]
    t_src, t_dst = tgt_edge_index[0], tgt_edge_index[1]

    x = _gat_layer(src_x, s_src, s_dst, We0, ase0, ade0, be0, 4, 16, True)
    x = _gat_layer(x, s_src, s_dst, We1, ase1, ade1, be1, 1, 16, False)
    x_pool = jnp.where(src_mask[:, None], -jnp.inf, x)
    z = jax.ops.segment_max(x_pool, src_batch, num_segments=B)

    dec = jnp.take(z, tgt_batch, axis=0)
    d = jnp.concatenate([dec, tgt_x], axis=1)
    d = _gat_layer(d, t_src, t_dst, Wd0, asd0, add0, bd0, 4, 16, True)
    d = jnp.concatenate([d, tgt_x], axis=1)
    d = _gat_layer(d, t_src, t_dst, Wd1, asd1, add1, bd1, 1, 16, False)
    return (z, d)


# speedup vs baseline: 8.3050x; 1.6159x over previous
import functools
import jax
import jax.numpy as jnp
from jax.experimental import pallas as pl

_N_BLK = 1000
_E_BLK = 8000


def _mm_body(x_ref, w_ref, o_ref):
    o_ref[...] = jnp.dot(x_ref[...], w_ref[...],
                         preferred_element_type=jnp.float32)


def _node_matmul(x, w):
    n, din = x.shape
    f = w.shape[1]
    grid = (pl.cdiv(n, _N_BLK),)
    return pl.pallas_call(
        _mm_body,
        grid=grid,
        in_specs=[
            pl.BlockSpec((_N_BLK, din), lambda i: (i, 0)),
            pl.BlockSpec((din, f), lambda i: (0, 0)),
        ],
        out_specs=pl.BlockSpec((_N_BLK, f), lambda i: (i, 0)),
        out_shape=jax.ShapeDtypeStruct((n, f), jnp.float32),
    )(x, w)


def _edge_body(hc, hs_ref, adst_ref, r_ref, o_ref):
    a = hs_ref[:, hc:] + adst_ref[...]
    a = jnp.where(a >= 0.0, a, 0.2 * a)
    ex = jnp.exp(a)
    o_ref[:, : a.shape[1]] = ex
    exrep = jnp.dot(ex, r_ref[...], preferred_element_type=jnp.float32)
    o_ref[:, a.shape[1]:] = exrep * hs_ref[:, :hc]


def _edge_coef(hs_g, adst_g, rmat):
    e, h = adst_g.shape
    hc = hs_g.shape[1] - h
    grid = (pl.cdiv(e, _E_BLK),)
    return pl.pallas_call(
        functools.partial(_edge_body, hc),
        grid=grid,
        in_specs=[
            pl.BlockSpec((_E_BLK, hc + h), lambda i: (i, 0)),
            pl.BlockSpec((_E_BLK, h), lambda i: (i, 0)),
            pl.BlockSpec((h, hc), lambda i: (0, 0)),
        ],
        out_specs=pl.BlockSpec((_E_BLK, h + hc), lambda i: (i, 0)),
        out_shape=jax.ShapeDtypeStruct((e, h + hc), jnp.float32),
    )(hs_g, adst_g, rmat)


def _epilogue_body(relu, accw_ref, den_ref, asrc_ref, adst_ref, h_ref,
                   r_ref, b_ref, o_ref):
    a = asrc_ref[...] + adst_ref[...]
    a = jnp.where(a >= 0.0, a, 0.2 * a)
    exl = jnp.exp(a)
    dtot = jnp.maximum(den_ref[...] + exl, 1e-16)
    inv = 1.0 / dtot
    r = r_ref[...]
    exlrep = jnp.dot(exl, r, preferred_element_type=jnp.float32)
    invrep = jnp.dot(inv, r, preferred_element_type=jnp.float32)
    out = (accw_ref[...] + exlrep * h_ref[...]) * invrep + b_ref[...]
    if relu:
        out = jnp.maximum(out, 0.0)
    o_ref[...] = out


def _epilogue(accw, den, asrc, adst, h, rmat, b, relu):
    n, hc = accw.shape
    nh = den.shape[1]
    grid = (pl.cdiv(n, _N_BLK),)
    return pl.pallas_call(
        functools.partial(_epilogue_body, relu),
        grid=grid,
        in_specs=[
            pl.BlockSpec((_N_BLK, hc), lambda i: (i, 0)),
            pl.BlockSpec((_N_BLK, nh), lambda i: (i, 0)),
            pl.BlockSpec((_N_BLK, nh), lambda i: (i, 0)),
            pl.BlockSpec((_N_BLK, nh), lambda i: (i, 0)),
            pl.BlockSpec((_N_BLK, hc), lambda i: (i, 0)),
            pl.BlockSpec((nh, hc), lambda i: (0, 0)),
            pl.BlockSpec((1, hc), lambda i: (0, 0)),
        ],
        out_specs=pl.BlockSpec((_N_BLK, hc), lambda i: (i, 0)),
        out_shape=jax.ShapeDtypeStruct((n, hc), jnp.float32),
    )(accw, den, asrc, adst, h, rmat, b)


def _gat_layer(x, src, dst, W, a_src, a_dst, b, H, C, relu):
    n = x.shape[0]
    hc = H * C
    eye = jnp.eye(H, dtype=jnp.float32)
    amat_s = (a_src[0][:, :, None] * eye[:, None, :]).reshape(hc, H)
    amat_d = (a_dst[0][:, :, None] * eye[:, None, :]).reshape(hc, H)
    rmat = jnp.repeat(eye, C, axis=1)
    wfull = jnp.concatenate([W, W @ amat_s, W @ amat_d], axis=1)
    hfull = _node_matmul(x, wfull)
    h = hfull[:, :hc]
    asrc = hfull[:, hc:hc + H]
    adst = hfull[:, hc + H:]
    hs_g = jnp.take(hfull[:, :hc + H], src, axis=0)
    adst_g = jnp.take(adst, dst, axis=0)
    exw = _edge_coef(hs_g, adst_g, rmat)
    seg = jax.ops.segment_sum(exw, dst, num_segments=n)
    den, accw = seg[:, :H], seg[:, H:]
    return _epilogue(accw, den, asrc, adst, h, rmat, b.reshape(1, hc), relu)


def kernel(src_x, src_edge_index, src_edge_mask, src_batch, src_mask,
           tgt_x, tgt_edge_index, tgt_edge_mask, tgt_batch,
           We0, ase0, ade0, be0, We1, ase1, ade1, be1,
           Wd0, asd0, add0, bd0, Wd1, asd1, add1, bd1):
    B = 64
    s_src, s_dst = src_edge_index[0], src_edge_index[1]
    t_src, t_dst = tgt_edge_index[0], tgt_edge_index[1]

    x = _gat_layer(src_x, s_src, s_dst, We0, ase0, ade0, be0, 4, 16, True)
    x = _gat_layer(x, s_src, s_dst, We1, ase1, ade1, be1, 1, 16, False)
    x_pool = jnp.where(src_mask[:, None], -jnp.inf, x)
    z = jax.ops.segment_max(x_pool, src_batch, num_segments=B)

    dec = jnp.take(z, tgt_batch, axis=0)
    d = jnp.concatenate([dec, tgt_x], axis=1)
    d = _gat_layer(d, t_src, t_dst, Wd0, asd0, add0, bd0, 4, 16, True)
    d = jnp.concatenate([d, tgt_x], axis=1)
    d = _gat_layer(d, t_src, t_dst, Wd1, asd1, add1, bd1, 1, 16, False)
    return (z, d)
